# baseline (speedup 1.0000x reference)
import functools

import jax
import jax.numpy as jnp
from jax import lax
from jax.experimental import pallas as pl
from jax.experimental.pallas import tpu as pltpu
from jax.experimental.pallas import tpu_sc as plsc

_VX = 0.16
_VY = 0.16
_XOFF = 0.16 / 2 + 0.0
_YOFF = 0.16 / 2 + (-39.68)
_XL = 432
_YL = 496
_CO = 64
_P = 12000
_NPTS = 32
_BS = 4

_BP = 240
_NBLK = _P // _BP
_CSTRIDE = _YL * _XL
_BSTRIDE = _CO * _CSTRIDE
_NWORDS = _BS * _BSTRIDE
_ZROWS = _NWORDS // 128
_ZBLK = 4464
_NW = 32
_SLOTS_W = 384
_ROWS_W = _SLOTS_W * _CO // 128


def _pfn_body(p_ref, aux_ref, m_ref, c_ref, cmax_ref, cmin_ref, sums_ref):
    i = pl.program_id(0)
    p3 = p_ref[...]
    pts = p3.reshape(_BP * _NPTS, 4)
    conv = jnp.dot(pts, m_ref[...], preferred_element_type=jnp.float32,
                   precision=lax.Precision.HIGHEST)
    psum = jnp.sum(p3, axis=1)
    auxv = aux_ref[...]
    inv_n = auxv[:, 2:3]
    u = jnp.concatenate(
        [auxv[:, 0:2], psum[:, 0:3] * inv_n,
         jnp.zeros((_BP, 3), jnp.float32)], axis=1)
    q = jnp.dot(u, c_ref[...], preferred_element_type=jnp.float32,
                precision=lax.Precision.HIGHEST)
    iota = lax.broadcasted_iota(jnp.int32, (_BP, _NPTS), 1).astype(jnp.float32)
    mask = (iota < auxv[:, 3:4]).astype(jnp.float32)
    conv3 = (conv.reshape(_BP, _NPTS, _CO) + q[:, None, :]) * mask[:, :, None]
    cmax_ref[...] = jnp.max(conv3, axis=1)
    cmin_ref[...] = jnp.min(conv3, axis=1)
    s = jnp.sum(conv3, axis=(0, 1))
    ss = jnp.sum(conv3 * conv3, axis=(0, 1))
    pad = jnp.zeros((64,), jnp.float32)
    upd = jnp.concatenate(
        [jnp.concatenate([s, pad])[None],
         jnp.concatenate([ss, pad])[None],
         jnp.zeros((6, 128), jnp.float32)], axis=0)

    @pl.when(i == 0)
    def _():
        sums_ref[...] = upd

    @pl.when(i > 0)
    def _():
        sums_ref[...] = sums_ref[...] + upd


def _zero_body(o_ref):
    o_ref[...] = jnp.zeros((_ZBLK, 128), jnp.float32)


def _sc_scatter_body(vidx_hbm, widx_hbm, cmm_hbm, sb_hbm,
                     canvas_hbm, vidx_v, rows_v,
                     widx_v, wdat_v, sb_v, gsem, ssem):
    wid = lax.axis_index("s") * 2 + lax.axis_index("c")
    pltpu.sync_copy(vidx_hbm.at[wid], vidx_v)
    pltpu.sync_copy(widx_hbm.at[wid], widx_v)
    pltpu.sync_copy(sb_hbm, sb_v)
    handles = []
    for c in range(3):
        handles.append(pltpu.async_copy(
            cmm_hbm.at[vidx_v.at[c]], rows_v.at[pl.ds(c * 128, 128)], gsem))
    for h in handles:
        h.wait()

    sc_k = [sb_v[0, pl.ds(16 * k, 16)] for k in range(4)]
    bi_k = [sb_v[1, pl.ds(16 * k, 16)] for k in range(4)]

    def rowbody(r, carry):
        p_a = 2 * r
        p_b = p_a + 1
        for k in range(8):
            kk = k & 3
            p = p_a if k < 4 else p_b
            vmax = rows_v[p, pl.ds(16 * kk, 16)]
            vmin = rows_v[p, pl.ds(64 + 16 * kk, 16)]
            v = jnp.maximum(sc_k[kk] * vmax, sc_k[kk] * vmin) + bi_k[kk]
            v = jnp.maximum(v, 0.0)
            wdat_v[r, pl.ds(16 * k, 16)] = v
        return carry

    lax.fori_loop(0, _ROWS_W, rowbody, 0)

    gk = 16

    def grp(g, carry):
        hs = []
        for t in range(gk):
            j = g * gk + t
            hs.append(pltpu.async_copy(
                wdat_v.at[j], canvas_hbm.at[widx_v.at[j]], ssem))
        for h in hs:
            h.wait()
        return carry

    lax.fori_loop(0, _ROWS_W // gk, grp, 0)


_SC_CACHE = []


def _get_sc_scatter():
    if not _SC_CACHE:
        _SC_CACHE.append(functools.partial(
            pl.kernel,
            out_type=(),
            mesh=plsc.VectorSubcoreMesh(core_axis_name="c",
                                        subcore_axis_name="s"),
            scratch_types=[
                pltpu.VMEM((3, 128), jnp.int32),
                pltpu.VMEM((_SLOTS_W, 128), jnp.float32),
                pltpu.VMEM((_ROWS_W, 128), jnp.int32),
                pltpu.VMEM((_ROWS_W, 128), jnp.float32),
                pltpu.VMEM((2, _CO), jnp.float32),
                pltpu.SemaphoreType.DMA,
                pltpu.SemaphoreType.DMA,
            ],
        )(_sc_scatter_body))
    return _SC_CACHE[0]


def kernel(pillars, coors_batch, npoints_per_pillar, W, gamma, beta):
    f32 = jnp.float32
    i32 = jnp.int32
    b = coors_batch[:, 0]
    gx = coors_batch[:, 1]
    gy = coors_batch[:, 2]
    cx = gx.astype(f32) * _VX + _XOFF
    cy = gy.astype(f32) * _VY + _YOFF
    npf = npoints_per_pillar.astype(f32)
    aux = jnp.stack([cx, cy, 1.0 / npf, npf,
                     jnp.zeros_like(npf), jnp.zeros_like(npf),
                     jnp.zeros_like(npf), jnp.zeros_like(npf)], axis=1)

    a0 = W[:, 0] + W[:, 7]
    a1 = W[:, 1] + W[:, 8]
    m_mat = jnp.stack([a0 + W[:, 4], a1 + W[:, 5], W[:, 2] + W[:, 6],
                       W[:, 3]], axis=0)
    zr = jnp.zeros_like(a0)
    c_mat = jnp.stack([-a0, -a1, -W[:, 4], -W[:, 5], -W[:, 6],
                       zr, zr, zr], axis=0)

    cmax, cmin, sums = pl.pallas_call(
        _pfn_body,
        grid=(_NBLK,),
        in_specs=[
            pl.BlockSpec((_BP, _NPTS, 4), lambda i: (i, 0, 0)),
            pl.BlockSpec((_BP, 8), lambda i: (i, 0)),
            pl.BlockSpec((4, _CO), lambda i: (0, 0)),
            pl.BlockSpec((8, _CO), lambda i: (0, 0)),
        ],
        out_specs=[
            pl.BlockSpec((_BP, _CO), lambda i: (i, 0)),
            pl.BlockSpec((_BP, _CO), lambda i: (i, 0)),
            pl.BlockSpec((8, 128), lambda i: (0, 0)),
        ],
        out_shape=[
            jax.ShapeDtypeStruct((_P, _CO), f32),
            jax.ShapeDtypeStruct((_P, _CO), f32),
            jax.ShapeDtypeStruct((8, 128), f32),
        ],
    )(pillars, aux, m_mat, c_mat)

    n_tot = float(_P * _NPTS)
    s = sums[0, :_CO]
    ss = sums[1, :_CO]
    mean = s / n_tot
    var = ss / n_tot - mean * mean
    inv_std = lax.rsqrt(var + 1e-3)
    scale = gamma * inv_std
    bias = beta - mean * scale
    sb = jnp.stack([scale, bias], axis=0)

    p_ids = jnp.arange(_P, dtype=i32)
    cell = (b * _XL + gx) * _YL + gy
    stamp = jnp.zeros((_BS * _XL * _YL,), i32).at[cell].max(p_ids + 1)
    win = stamp[cell] - 1
    basew = b * _BSTRIDE + gy * _XL + gx
    slot_p = jnp.minimum(jnp.arange(_NW * _SLOTS_W, dtype=i32), _P - 1)
    vidx_slot = win[slot_p].reshape(_NW, 3, 128)
    widx_slot = (basew[slot_p][:, None]
                 + jnp.arange(_CO, dtype=i32)[None, :] * _CSTRIDE
                 ).reshape(_NW, _ROWS_W, 128)

    canvas0 = pl.pallas_call(
        _zero_body,
        grid=(_ZROWS // _ZBLK,),
        out_specs=pl.BlockSpec((_ZBLK, 128), lambda i: (i, 0)),
        out_shape=jax.ShapeDtypeStruct((_ZROWS, 128), f32),
    )()

    cmm = jnp.concatenate([cmax, cmin], axis=1)
    cref = jax.new_ref(canvas0.reshape(_NWORDS))
    _get_sc_scatter()(vidx_slot, widx_slot, cmm, sb, cref)
    return cref[...].reshape(_BS, _CO, _YL, _XL)
